# baseline (speedup 1.0000x reference)
import functools

import jax
import jax.numpy as jnp
from jax import lax
from jax.experimental import pallas as pl
from jax.experimental.pallas import tpu as pltpu
from jax.experimental.pallas import tpu_sc as plsc

B = 4096
S = 200
D = 64
O = 20

NC = 2
NS = 16
NW = NC * NS
ROWS_PER_W = B // NW
ROWS_PER_SC = B // NC
CHUNK = 100
CHUNKS_PER_ROW = S // CHUNK
CHUNKS_PER_W = ROWS_PER_W * CHUNKS_PER_ROW
N_CHUNKS = B * CHUNKS_PER_ROW


def _sc_pool(x2, dest2, table):
    mesh = plsc.VectorSubcoreMesh(core_axis_name="c", subcore_axis_name="s")

    @functools.partial(
        pl.kernel,
        out_type=jax.ShapeDtypeStruct((B, D), jnp.float32),
        mesh=mesh,
        scratch_types=[
            pltpu.VMEM((CHUNKS_PER_W, CHUNK), jnp.int32),
            pltpu.VMEM((CHUNKS_PER_W, CHUNK), jnp.int32),
            pltpu.VMEM((CHUNK, D), jnp.float32),
            pltpu.VMEM((ROWS_PER_W, D), jnp.float32),
            pltpu.VMEM_SHARED((ROWS_PER_SC, D), jnp.float32),
            pltpu.SemaphoreType.DMA,
        ],
        compiler_params=pltpu.CompilerParams(use_tc_tiling_on_sc=False),
    )
    def body(x_hbm, dest_hbm, table_hbm, out_hbm,
             idx_v, dest_v, buf, zbuf, acc_sh, sem):
        c = lax.axis_index("c")
        s = lax.axis_index("s")
        wid = c * NS + s
        base_chunk = wid * CHUNKS_PER_W
        local_base = s * ROWS_PER_W

        pltpu.sync_copy(x_hbm.at[pl.ds(base_chunk, CHUNKS_PER_W)], idx_v)
        pltpu.sync_copy(dest_hbm.at[pl.ds(base_chunk, CHUNKS_PER_W)], dest_v)

        zero = jnp.zeros((16,), jnp.float32)

        def zero_body(i, carry):
            r = i // (D // 16)
            j = i % (D // 16)
            zbuf[r, pl.ds(j * 16, 16)] = zero
            return carry

        lax.fori_loop(0, ROWS_PER_W * (D // 16), zero_body, 0)
        pltpu.sync_copy(zbuf, acc_sh.at[pl.ds(local_base, ROWS_PER_W)])

        def chunk_body(k, carry):
            pltpu.async_copy(table_hbm.at[idx_v.at[k]], buf, sem).wait()
            pltpu.sync_copy(buf, acc_sh.at[dest_v.at[k]], add=True)
            return carry

        lax.fori_loop(0, CHUNKS_PER_W, chunk_body, 0)

        pltpu.sync_copy(
            acc_sh.at[pl.ds(local_base, ROWS_PER_W)],
            out_hbm.at[pl.ds(wid * ROWS_PER_W, ROWS_PER_W)],
        )

    return body(x2, dest2, table)


def _head_body(s_ref, w_ref, b_ref, o_ref):
    rep = jnp.maximum(s_ref[...] * (1.0 / S), 0.0)
    o_ref[...] = (
        jnp.dot(rep, w_ref[...], preferred_element_type=jnp.float32)
        + b_ref[...]
    )


def _tc_head(sums, W, b):
    blk = 1024
    return pl.pallas_call(
        _head_body,
        out_shape=jax.ShapeDtypeStruct((B, O), jnp.float32),
        grid=(B // blk,),
        in_specs=[
            pl.BlockSpec((blk, D), lambda i: (i, 0)),
            pl.BlockSpec((D, O), lambda i: (0, 0)),
            pl.BlockSpec((1, O), lambda i: (0, 0)),
        ],
        out_specs=pl.BlockSpec((blk, O), lambda i: (i, 0)),
    )(sums, W, b.reshape(1, O))


def kernel(x, lengths, table, W, b):
    del lengths
    x2 = x.astype(jnp.int32).reshape(N_CHUNKS, CHUNK)
    dest = (jnp.arange(N_CHUNKS, dtype=jnp.int32) // CHUNKS_PER_ROW) % ROWS_PER_SC
    dest2 = jnp.broadcast_to(dest[:, None], (N_CHUNKS, CHUNK))
    sums = _sc_pool(x2, dest2, table)
    return _tc_head(sums, W, b)


# speedup vs baseline: 1.1549x; 1.1549x over previous
import functools

import jax
import jax.numpy as jnp
from jax import lax
from jax.experimental import pallas as pl
from jax.experimental.pallas import tpu as pltpu
from jax.experimental.pallas import tpu_sc as plsc

B = 4096
S = 200
D = 64
O = 20

NC = 2
NS = 16
NW = NC * NS
ROWS_PER_W = B // NW
ROWS_PER_SC = B // NC
CHUNK = 100
CHUNKS_PER_ROW = S // CHUNK
CHUNKS_PER_W = ROWS_PER_W * CHUNKS_PER_ROW
N_CHUNKS = B * CHUNKS_PER_ROW
PAIRS_PER_W = CHUNKS_PER_W // 2


def _sc_pool(xa, xb, dest2, table):
    mesh = plsc.VectorSubcoreMesh(core_axis_name="c", subcore_axis_name="s")

    @functools.partial(
        pl.kernel,
        out_type=jax.ShapeDtypeStruct((B, D), jnp.float32),
        mesh=mesh,
        scratch_types=[
            pltpu.VMEM((ROWS_PER_W, CHUNK), jnp.int32),
            pltpu.VMEM((ROWS_PER_W, CHUNK), jnp.int32),
            pltpu.VMEM((CHUNKS_PER_W, CHUNK), jnp.int32),
            pltpu.VMEM((CHUNK, D), jnp.float32),
            pltpu.VMEM((CHUNK, D), jnp.float32),
            pltpu.VMEM((ROWS_PER_W, D), jnp.float32),
            pltpu.VMEM_SHARED((ROWS_PER_SC, D), jnp.float32),
            pltpu.SemaphoreType.DMA,
            pltpu.SemaphoreType.DMA,
            pltpu.SemaphoreType.DMA,
            pltpu.SemaphoreType.DMA,
        ],
        compiler_params=pltpu.CompilerParams(use_tc_tiling_on_sc=False),
    )
    def body(xa_hbm, xb_hbm, dest_hbm, table_hbm, out_hbm,
             idx_a, idx_b, dest_v, buf0, buf1, zbuf, acc_sh, g0, g1, s0, s1):
        c = lax.axis_index("c")
        s = lax.axis_index("s")
        wid = c * NS + s
        base_chunk = wid * CHUNKS_PER_W
        local_base = s * ROWS_PER_W

        rows = pl.ds(wid * ROWS_PER_W, ROWS_PER_W)
        pltpu.sync_copy(xa_hbm.at[rows], idx_a)
        pltpu.sync_copy(xb_hbm.at[rows], idx_b)
        pltpu.sync_copy(dest_hbm.at[pl.ds(base_chunk, CHUNKS_PER_W)], dest_v)

        zero = jnp.zeros((16,), jnp.float32)

        def zero_body(i, carry):
            r = i // (D // 16)
            j = i % (D // 16)
            zbuf[r, pl.ds(j * 16, 16)] = zero
            return carry

        lax.fori_loop(0, ROWS_PER_W * (D // 16), zero_body, 0)
        pltpu.sync_copy(zbuf, acc_sh.at[pl.ds(local_base, ROWS_PER_W)])

        def start_gather(row, half, buf, sem):
            iv = idx_a if half == 0 else idx_b
            return pltpu.async_copy(table_hbm.at[iv.at[row]], buf, sem)

        def wait_gather(row, half, buf, sem):
            iv = idx_a if half == 0 else idx_b
            pltpu.make_async_copy(table_hbm.at[iv.at[row]], buf, sem).wait()

        def start_scatter(k, buf, sem):
            return pltpu.async_copy(
                buf, acc_sh.at[dest_v.at[k]], sem, add=True)

        def wait_scatter(k, buf, sem):
            pltpu.make_async_copy(
                buf, acc_sh.at[dest_v.at[k]], sem).wait()

        start_gather(0, 0, buf0, g0)

        def pair_body(kk, carry):
            k0 = 2 * kk

            @pl.when(kk > 0)
            def _():
                wait_scatter(k0 - 1, buf1, s1)

            start_gather(kk, 1, buf1, g1)
            wait_gather(kk, 0, buf0, g0)
            start_scatter(k0, buf0, s0)
            wait_scatter(k0, buf0, s0)
            nxt = jnp.minimum(kk + 1, PAIRS_PER_W - 1)
            start_gather(nxt, 0, buf0, g0)
            wait_gather(kk, 1, buf1, g1)
            start_scatter(k0 + 1, buf1, s1)
            return carry

        lax.fori_loop(0, PAIRS_PER_W, pair_body, 0)
        wait_scatter(CHUNKS_PER_W - 1, buf1, s1)
        wait_gather(PAIRS_PER_W - 1, 0, buf0, g0)

        pltpu.sync_copy(
            acc_sh.at[pl.ds(local_base, ROWS_PER_W)],
            out_hbm.at[pl.ds(wid * ROWS_PER_W, ROWS_PER_W)],
        )

    return body(xa, xb, dest2, table)


def _head_body(s_ref, w_ref, b_ref, o_ref):
    rep = jnp.maximum(s_ref[...] * (1.0 / S), 0.0)
    o_ref[...] = (
        jnp.dot(rep, w_ref[...], preferred_element_type=jnp.float32)
        + b_ref[...]
    )


def _tc_head(sums, W, b):
    blk = 1024
    return pl.pallas_call(
        _head_body,
        out_shape=jax.ShapeDtypeStruct((B, O), jnp.float32),
        grid=(B // blk,),
        in_specs=[
            pl.BlockSpec((blk, D), lambda i: (i, 0)),
            pl.BlockSpec((D, O), lambda i: (0, 0)),
            pl.BlockSpec((1, O), lambda i: (0, 0)),
        ],
        out_specs=pl.BlockSpec((blk, O), lambda i: (i, 0)),
    )(sums, W, b.reshape(1, O))


def kernel(x, lengths, table, W, b):
    del lengths
    dest = (jnp.arange(N_CHUNKS, dtype=jnp.int32) // CHUNKS_PER_ROW) % ROWS_PER_SC
    dest2 = jnp.broadcast_to(dest[:, None], (N_CHUNKS, CHUNK))
    xi = x.astype(jnp.int32)
    sums = _sc_pool(xi[:, :CHUNK], xi[:, CHUNK:], dest2, table)
    return _tc_head(sums, W, b)
